# baseline (speedup 1.0000x reference)
import functools

import jax
import jax.numpy as jnp
from jax import lax
from jax.experimental import pallas as pl
from jax.experimental.pallas import tpu as pltpu
from jax.experimental.pallas import tpu_sc as plsc

_K = 8


def _topk_body(x_ref, wq_ref, wk_ref, wv_ref, vt_ref, fidx_ref):
    c, n = x_ref.shape[1], x_ref.shape[2]
    b = pl.program_id(0)
    xb = x_ref[0]

    def nt_dot(a, bb):
        return lax.dot_general(a, bb, (((1,), (1,)), ((), ())),
                               preferred_element_type=jnp.float32)

    q = nt_dot(xb, wq_ref[...])
    k = nt_dot(xb, wk_ref[...])
    vtb = nt_dot(wv_ref[...], xb)
    cpad = vt_ref.shape[2] - vtb.shape[1]
    vt_ref[0] = jnp.concatenate(
        [vtb, jnp.zeros((vtb.shape[0], cpad), jnp.float32)], axis=1)

    ks = jnp.sqrt(jnp.sum(k * k, axis=0, keepdims=True))
    kn = k / jnp.maximum(ks, 1e-12)
    qs = jnp.sqrt(jnp.sum(q * q, axis=0, keepdims=True))
    qn = q / jnp.maximum(qs, 1e-12)

    sim = lax.dot_general(kn, qn, (((0,), (0,)), ((), ())),
                          preferred_element_type=jnp.float32)
    sim = jnp.maximum(sim, 0.0)

    iota_m = lax.broadcasted_iota(jnp.int32, (n, n), 1)
    for kk in range(_K):
        mx = jnp.max(sim, axis=1, keepdims=True)
        am = jnp.min(jnp.where(sim == mx, iota_m, n), axis=1,
                     keepdims=True)
        if kk + 1 < _K:
            sim = jnp.where(iota_m == am, -1.0, sim)
        fidx_ref[0, :, kk:kk + 1] = am + b * n


def _conv_body(prime_ref, w3_ref, bias_ref, wo_ref, out_ref):
    n, c = prime_ref.shape[1], w3_ref.shape[2]
    xct = jnp.zeros((n, c), jnp.float32)
    for kk in range(_K):
        xct = xct + lax.dot_general(prime_ref[0, :, kk, :], w3_ref[kk],
                                    (((1,), (0,)), ((), ())),
                                    preferred_element_type=jnp.float32)
    xct = xct + bias_ref[...]
    out_ref[0] = lax.dot_general(xct, wo_ref[...], (((0,), (1,)), ((), ())),
                                 preferred_element_type=jnp.float32)


def _sc_gather(vt, fidx, c):
    r = fidx.shape[0]
    ncores, nsub = 2, 16
    nw = ncores * nsub
    per_w = r // nw
    gw = 128
    nwin = per_w // gw
    mesh = plsc.VectorSubcoreMesh(core_axis_name="c", subcore_axis_name="s")

    @functools.partial(
        pl.kernel, mesh=mesh,
        out_type=jax.ShapeDtypeStruct((r, c), jnp.float32),
        scratch_types=[
            pltpu.VMEM((gw,), jnp.int32),
            pltpu.VMEM((gw, c), jnp.float32),
            pltpu.SemaphoreType.DMA,
        ],
    )
    def sck(vt_hbm, idx_hbm, out_hbm, idx_v, rows_v, sem):
        wid = lax.axis_index("s") * ncores + lax.axis_index("c")
        base = wid * per_w

        def body(i, carry):
            start = base + i * gw
            pltpu.sync_copy(idx_hbm.at[pl.ds(start, gw)], idx_v)
            pltpu.async_copy(vt_hbm.at[idx_v], rows_v, sem).wait()
            pltpu.sync_copy(rows_v, out_hbm.at[pl.ds(start, gw)])
            return carry

        lax.fori_loop(0, nwin, body, 0)

    return sck(vt, fidx)


def kernel(x, Wq, Wk, Wv, Wo, conv_w, conv_b):
    b, c, h, w = x.shape
    n = h * w
    xf = x.reshape(b, c, n)
    cp = 256
    w3 = jnp.concatenate(
        [conv_w.transpose(2, 1, 0),
         jnp.zeros((conv_w.shape[2], cp - c, c), conv_w.dtype)], axis=1)
    bias = conv_b.reshape(1, c)

    full = lambda shp: pl.BlockSpec(shp, lambda i: tuple(0 for _ in shp))
    vt, fidx = pl.pallas_call(
        _topk_body,
        grid=(b,),
        in_specs=[
            pl.BlockSpec((1, c, n), lambda i: (i, 0, 0)),
            full((n, n)), full((n, n)), full((n, n)),
        ],
        out_specs=[
            pl.BlockSpec((1, n, cp), lambda i: (i, 0, 0)),
            pl.BlockSpec((1, n, _K), lambda i: (i, 0, 0)),
        ],
        out_shape=[
            jax.ShapeDtypeStruct((b, n, cp), jnp.float32),
            jax.ShapeDtypeStruct((b, n, _K), jnp.int32),
        ],
    )(xf, Wq, Wk, Wv)

    prime = _sc_gather(vt.reshape(b * n, cp), fidx.reshape(b * n * _K), cp)

    out = pl.pallas_call(
        _conv_body,
        grid=(b,),
        in_specs=[
            pl.BlockSpec((1, n, _K, cp), lambda i: (i, 0, 0, 0)),
            full((_K, cp, c)),
            full((1, c)),
            full((n, n)),
        ],
        out_specs=pl.BlockSpec((1, c, n), lambda i: (i, 0, 0)),
        out_shape=jax.ShapeDtypeStruct((b, c, n), jnp.float32),
    )(prime.reshape(b, n, _K, cp), w3, bias, Wo)
    return out.reshape(b, c, h, w)
